# baseline (speedup 1.0000x reference)
import jax
import jax.numpy as jnp
from jax.experimental import pallas as pl
from jax.experimental.pallas import tpu as pltpu

_IN = 16
_OUT = 7
_TILE = 8192


def _dual_stream_kernel(lo_ref, hi_ref, w_ref, o_ref):
    w = w_ref[...]
    o_ref[0] = jnp.dot(lo_ref[...], w, preferred_element_type=jnp.float32)
    o_ref[1] = jnp.dot(hi_ref[...], w, preferred_element_type=jnp.float32)


def kernel(x, w):
    n, in_feats = x.shape
    assert in_feats == _IN and w.shape == (_IN, _OUT)
    assert n % (2 * _TILE) == 0
    half = n // 2
    steps = half // _TILE
    hi_base = steps

    y2 = pl.pallas_call(
        _dual_stream_kernel,
        out_shape=jax.ShapeDtypeStruct((2, half, _OUT), x.dtype),
        grid=(steps,),
        in_specs=[
            pl.BlockSpec((_TILE, _IN), lambda i: (i, 0)),
            pl.BlockSpec((_TILE, _IN), lambda i: (i + hi_base, 0)),
            pl.BlockSpec((_IN, _OUT), lambda i: (0, 0)),
        ],
        out_specs=pl.BlockSpec((2, _TILE, _OUT), lambda i: (0, i, 0)),
        compiler_params=pltpu.CompilerParams(
            dimension_semantics=("parallel",),
        ),
        cost_estimate=pl.CostEstimate(
            flops=2 * n * _IN * _OUT,
            transcendentals=0,
            bytes_accessed=(n * (_IN + _OUT) + _IN * _OUT) * x.dtype.itemsize,
        ),
    )(x, x, w)

    return y2.reshape(n, _OUT)
